# baseline (speedup 1.0000x reference)
import functools

import jax
import jax.numpy as jnp
from jax import lax
from jax.experimental import pallas as pl
from jax.experimental.pallas import tpu as pltpu
from jax.experimental.pallas import tpu_sc as plsc

V = 100000
D = 64
B = 1024
L = 20

NUM_CORES = 2
NUM_SUBCORES = 16
NW = NUM_CORES * NUM_SUBCORES
BPW = B // NW
IDX_PER_W = BPW * L
CHUNK = 128
NCHUNK = IDX_PER_W // CHUNK
LANES = 16

_sc_mesh = plsc.VectorSubcoreMesh(core_axis_name="c", subcore_axis_name="s")


@functools.partial(
    pl.kernel,
    mesh=_sc_mesh,
    out_type=jax.ShapeDtypeStruct((B, D), jnp.float32),
    scratch_types=[
        pltpu.VMEM((IDX_PER_W,), jnp.int32),
        pltpu.VMEM((IDX_PER_W, D), jnp.float32),
        pltpu.VMEM((BPW, D), jnp.float32),
        pltpu.SemaphoreType.DMA,
    ],
    compiler_params=pltpu.CompilerParams(use_tc_tiling_on_sc=False),
)
def _pool_sc(idx_hbm, table_hbm, out_hbm, idx_v, rows_v, pool_v, sem):
    wid = lax.axis_index("s") * NUM_CORES + lax.axis_index("c")
    base = wid * IDX_PER_W
    pltpu.sync_copy(idx_hbm.at[pl.ds(base, IDX_PER_W)], idx_v)
    copies = []
    for c in range(NCHUNK):
        copies.append(
            pltpu.async_copy(
                table_hbm.at[idx_v.at[pl.ds(c * CHUNK, CHUNK)]],
                rows_v.at[pl.ds(c * CHUNK, CHUNK)],
                sem,
            )
        )
    for cp in copies:
        cp.wait()

    scale = jnp.float32(1.0 / L)

    def body(b_i, carry):
        row0 = b_i * L
        for d in range(D // LANES):
            sl = pl.ds(d * LANES, LANES)
            acc = rows_v[row0, sl]
            for l_i in range(1, L):
                acc = acc + rows_v[row0 + l_i, sl]
            pool_v[b_i, sl] = acc * scale
        return carry

    lax.fori_loop(0, BPW, body, 0)
    pltpu.sync_copy(pool_v, out_hbm.at[pl.ds(wid * BPW, BPW)])


VBLK = 12544
BBLK = 128
_VGRID = pl.cdiv(V, VBLK)
_BGRID = B // BBLK


def _decode_body(p_ref, w_ref, bias_ref, o_ref):
    o_ref[...] = (
        lax.dot_general(
            p_ref[...],
            w_ref[...],
            (((1,), (1,)), ((), ())),
            preferred_element_type=jnp.float32,
        )
        + bias_ref[...]
    )


_decode = pl.pallas_call(
    _decode_body,
    grid=(_VGRID, _BGRID),
    in_specs=[
        pl.BlockSpec((BBLK, D), lambda i, j: (j, 0)),
        pl.BlockSpec((VBLK, D), lambda i, j: (i, 0)),
        pl.BlockSpec((1, VBLK), lambda i, j: (0, i)),
    ],
    out_specs=pl.BlockSpec((BBLK, VBLK), lambda i, j: (j, i)),
    out_shape=jax.ShapeDtypeStruct((B, V), jnp.float32),
)


def kernel(contextTsr, emb_table, W, b):
    idx_flat = contextTsr.reshape(-1).astype(jnp.int32)
    pooled = _pool_sc(idx_flat, emb_table)
    return _decode(pooled, W, b.reshape(1, V))


# speedup vs baseline: 2.8008x; 2.8008x over previous
import functools

import jax
import jax.numpy as jnp
from jax import lax
from jax.experimental import pallas as pl
from jax.experimental.pallas import tpu as pltpu
from jax.experimental.pallas import tpu_sc as plsc

V = 100000
D = 64
B = 1024
L = 20

NUM_CORES = 2
NUM_SUBCORES = 16
NW = NUM_CORES * NUM_SUBCORES
BPW = B // NW
IDX_PER_W = BPW * L
CHUNK = 128
NCHUNK = IDX_PER_W // CHUNK
LANES = 16

_sc_mesh = plsc.VectorSubcoreMesh(core_axis_name="c", subcore_axis_name="s")


@functools.partial(
    pl.kernel,
    mesh=_sc_mesh,
    out_type=jax.ShapeDtypeStruct((B, D), jnp.float32),
    scratch_types=[
        pltpu.VMEM((IDX_PER_W,), jnp.int32),
        pltpu.VMEM((IDX_PER_W, D), jnp.float32),
        pltpu.VMEM((BPW, D), jnp.float32),
        pltpu.SemaphoreType.DMA,
    ],
    compiler_params=pltpu.CompilerParams(use_tc_tiling_on_sc=False),
)
def _pool_sc(idx_hbm, table_hbm, out_hbm, idx_v, rows_v, pool_v, sem):
    wid = lax.axis_index("s") * NUM_CORES + lax.axis_index("c")
    base = wid * IDX_PER_W
    pltpu.sync_copy(idx_hbm.at[pl.ds(base, IDX_PER_W)], idx_v)
    copies = []
    for c in range(NCHUNK):
        copies.append(
            pltpu.async_copy(
                table_hbm.at[idx_v.at[pl.ds(c * CHUNK, CHUNK)]],
                rows_v.at[pl.ds(c * CHUNK, CHUNK)],
                sem,
            )
        )
    for cp in copies:
        cp.wait()

    scale = jnp.float32(1.0 / L)

    def body(b_i, carry):
        row0 = b_i * L
        for d in range(D // LANES):
            sl = pl.ds(d * LANES, LANES)
            acc = rows_v[row0, sl]
            for l_i in range(1, L):
                acc = acc + rows_v[row0 + l_i, sl]
            pool_v[b_i, sl] = acc * scale
        return carry

    lax.fori_loop(0, BPW, body, 0)
    pltpu.sync_copy(pool_v, out_hbm.at[pl.ds(wid * BPW, BPW)])


VBLK = 2048
_VGRID = pl.cdiv(V, VBLK)


def _decode_body(p_ref, wt_ref, bias_ref, o_ref):
    acc = lax.dot_general(
        wt_ref[...],
        p_ref[...],
        (((0,), (1,)), ((), ())),
        preferred_element_type=jnp.float32,
    )
    ones = jnp.ones((1, B), jnp.float32)
    bias2d = lax.dot_general(
        bias_ref[...],
        ones,
        (((0,), (0,)), ((), ())),
        preferred_element_type=jnp.float32,
    )
    o_ref[...] = acc + bias2d


_decode = pl.pallas_call(
    _decode_body,
    grid=(_VGRID,),
    in_specs=[
        pl.BlockSpec((B, D), lambda i: (0, 0)),
        pl.BlockSpec((D, VBLK), lambda i: (0, i)),
        pl.BlockSpec((1, VBLK), lambda i: (0, i)),
    ],
    out_specs=pl.BlockSpec((VBLK, B), lambda i: (i, 0)),
    out_shape=jax.ShapeDtypeStruct((V, B), jnp.float32),
)


def kernel(contextTsr, emb_table, W, b):
    idx_flat = contextTsr.reshape(-1).astype(jnp.int32)
    pooled = _pool_sc(idx_flat, emb_table)
    out_t = _decode(pooled, W.T, b.reshape(1, V))
    return out_t.T


# speedup vs baseline: 3.6220x; 1.2932x over previous
import functools

import jax
import jax.numpy as jnp
from jax import lax
from jax.experimental import pallas as pl
from jax.experimental.pallas import tpu as pltpu
from jax.experimental.pallas import tpu_sc as plsc

V = 100000
D = 64
B = 1024
L = 20

NUM_CORES = 2
NUM_SUBCORES = 16
NW = NUM_CORES * NUM_SUBCORES
DPW = D // NW
LANES = 16
_BGROUPS = B // LANES

_sc_mesh = plsc.VectorSubcoreMesh(core_axis_name="c", subcore_axis_name="s")


@functools.partial(
    pl.kernel,
    mesh=_sc_mesh,
    out_type=jax.ShapeDtypeStruct((D, B), jnp.float32),
    scratch_types=[
        pltpu.VMEM((L, B), jnp.int32),
        pltpu.VMEM((V,), jnp.float32),
        pltpu.VMEM((B,), jnp.float32),
        pltpu.SemaphoreType.DMA,
    ],
    compiler_params=pltpu.CompilerParams(needs_layout_passes=False),
)
def _pool_sc(idx_hbm, table_hbm, out_hbm, idx_v, slab_v, pool_v, sem):
    wid = lax.axis_index("s") * NUM_CORES + lax.axis_index("c")
    pltpu.sync_copy(idx_hbm, idx_v)
    scale = jnp.float32(1.0 / L)
    for d_off in range(DPW):
        d_row = wid * DPW + d_off
        pltpu.sync_copy(table_hbm.at[d_row], slab_v)

        def body(g, carry):
            sl = pl.ds(g * LANES, LANES)
            acc = jnp.zeros((LANES,), jnp.float32)
            for l_i in range(L):
                acc = acc + plsc.load_gather(slab_v, [idx_v[l_i, sl]])
            pool_v[sl] = acc * scale
            return carry

        lax.fori_loop(0, _BGROUPS, body, 0)
        pltpu.sync_copy(pool_v, out_hbm.at[d_row])


VBLK = 2048
_VGRID = pl.cdiv(V, VBLK)


def _decode_body(p_ref, wt_ref, bias_ref, o_ref):
    acc = lax.dot_general(
        wt_ref[...],
        p_ref[...],
        (((0,), (0,)), ((), ())),
        preferred_element_type=jnp.float32,
    )
    ones = jnp.ones((1, B), jnp.float32)
    bias2d = lax.dot_general(
        bias_ref[...],
        ones,
        (((0,), (0,)), ((), ())),
        preferred_element_type=jnp.float32,
    )
    o_ref[...] = acc + bias2d


_decode = pl.pallas_call(
    _decode_body,
    grid=(_VGRID,),
    in_specs=[
        pl.BlockSpec((D, B), lambda i: (0, 0)),
        pl.BlockSpec((D, VBLK), lambda i: (0, i)),
        pl.BlockSpec((1, VBLK), lambda i: (0, i)),
    ],
    out_specs=pl.BlockSpec((VBLK, B), lambda i: (i, 0)),
    out_shape=jax.ShapeDtypeStruct((V, B), jnp.float32),
)


def kernel(contextTsr, emb_table, W, b):
    pooled_t = _pool_sc(contextTsr.T.astype(jnp.int32), emb_table.T)
    out_t = _decode(pooled_t, W.T, b.reshape(1, V))
    return out_t.T


# speedup vs baseline: 3.6777x; 1.0154x over previous
import functools

import jax
import jax.numpy as jnp
from jax import lax
from jax.experimental import pallas as pl
from jax.experimental.pallas import tpu as pltpu
from jax.experimental.pallas import tpu_sc as plsc

V = 100000
D = 64
B = 1024
L = 20

NUM_CORES = 2
NUM_SUBCORES = 16
NW = NUM_CORES * NUM_SUBCORES
DPW = D // NW
LANES = 16
_BGROUPS = B // LANES

_sc_mesh = plsc.VectorSubcoreMesh(core_axis_name="c", subcore_axis_name="s")


@functools.partial(
    pl.kernel,
    mesh=_sc_mesh,
    out_type=jax.ShapeDtypeStruct((D, B), jnp.float32),
    scratch_types=[
        pltpu.VMEM((L, B), jnp.int32),
        pltpu.VMEM((V,), jnp.float32),
        pltpu.VMEM((B,), jnp.float32),
        pltpu.SemaphoreType.DMA,
    ],
    compiler_params=pltpu.CompilerParams(needs_layout_passes=False),
)
def _pool_sc(idx_hbm, table_hbm, out_hbm, idx_v, slab_v, pool_v, sem):
    wid = lax.axis_index("s") * NUM_CORES + lax.axis_index("c")
    idx_cp = pltpu.async_copy(idx_hbm, idx_v, sem)
    scale = jnp.float32(1.0 / L)
    for d_off in range(DPW):
        d_row = wid * DPW + d_off
        pltpu.sync_copy(table_hbm.at[d_row], slab_v)
        if d_off == 0:
            idx_cp.wait()

        def body(g, carry):
            sl = pl.ds(g * LANES, LANES)
            acc = jnp.zeros((LANES,), jnp.float32)
            for l_i in range(L):
                acc = acc + plsc.load_gather(slab_v, [idx_v[l_i, sl]])
            pool_v[sl] = acc * scale
            return carry

        lax.fori_loop(0, _BGROUPS, body, 0)
        pltpu.sync_copy(pool_v, out_hbm.at[d_row])


VBLK = 4096
_VGRID = pl.cdiv(V, VBLK)


def _decode_body(p_ref, wt_ref, bias_ref, o_ref):
    acc = lax.dot_general(
        wt_ref[...],
        p_ref[...],
        (((0,), (0,)), ((), ())),
        preferred_element_type=jnp.float32,
    )
    ones = jnp.ones((1, B), jnp.float32)
    bias2d = lax.dot_general(
        bias_ref[...],
        ones,
        (((0,), (0,)), ((), ())),
        preferred_element_type=jnp.float32,
    )
    o_ref[...] = acc + bias2d


_decode = pl.pallas_call(
    _decode_body,
    grid=(_VGRID,),
    in_specs=[
        pl.BlockSpec((D, B), lambda i: (0, 0)),
        pl.BlockSpec((D, VBLK), lambda i: (0, i)),
        pl.BlockSpec((1, VBLK), lambda i: (0, i)),
    ],
    out_specs=pl.BlockSpec((VBLK, B), lambda i: (i, 0)),
    out_shape=jax.ShapeDtypeStruct((V, B), jnp.float32),
)


def kernel(contextTsr, emb_table, W, b):
    pooled_t = _pool_sc(contextTsr.T.astype(jnp.int32), emb_table.T)
    out_t = _decode(pooled_t, W.T, b.reshape(1, V))
    return out_t.T
